# baseline (speedup 1.0000x reference)
import functools

import jax
import jax.numpy as jnp
from jax import lax
from jax.experimental import pallas as pl
from jax.experimental.pallas import tpu as pltpu
from jax.experimental.pallas import tpu_sc as plsc

N = 10000
E = 320000
NPAD = 10240
NC = 2
NS = 16
NW = NC * NS
K = 128
CH = 80
EPAD = NW * CH * K
ZROWS = NPAD // NS
DEGW = 16
NBUF = 8

MBLK = 512


def _make_agg(d, nhalves, with_deg, nbuf):
    mesh = plsc.VectorSubcoreMesh(core_axis_name="c", subcore_axis_name="s")

    out_type = [jax.ShapeDtypeStruct((NC, NPAD, d), jnp.float32)
                for _ in range(nhalves)]
    scratch = [
        pltpu.VMEM((CH, K), jnp.int32),
        pltpu.VMEM((CH, K), jnp.int32),
        [pltpu.VMEM((K, d), jnp.float32) for _ in range(nbuf)],
        [pltpu.SemaphoreType.DMA for _ in range(nbuf)],
        [pltpu.SemaphoreType.DMA for _ in range(nbuf)],
        pltpu.VMEM_SHARED((NPAD, d), jnp.float32),
    ]
    if with_deg:
        out_type.append(jax.ShapeDtypeStruct((NC, NPAD, DEGW), jnp.float32))
        scratch += [
            pltpu.VMEM((K, DEGW), jnp.float32),
            pltpu.VMEM((K, DEGW), jnp.float32),
            [pltpu.SemaphoreType.DMA for _ in range(nbuf)],
            pltpu.VMEM_SHARED((NPAD, DEGW), jnp.float32),
        ]

    @functools.partial(
        pl.kernel,
        out_type=out_type,
        mesh=mesh,
        compiler_params=pltpu.CompilerParams(use_tc_tiling_on_sc=False),
        scratch_types=scratch,
    )
    def agg(*refs):
        i = 0
        y_hbms = refs[i:i + nhalves]; i += nhalves
        src_hbm = refs[i]; i += 1
        dst_hbm = refs[i]; i += 1
        out_hbms = refs[i:i + nhalves]; i += nhalves
        if with_deg:
            outd_hbm = refs[i]; i += 1
        src_v = refs[i]; i += 1
        dst_v = refs[i]; i += 1
        rows = refs[i]; i += 1
        gsem = refs[i]; i += 1
        ssem = refs[i]; i += 1
        acc = refs[i]; i += 1
        if with_deg:
            ones_v = refs[i]; i += 1
            zb = refs[i]; i += 1
            dsem = refs[i]; i += 1
            accd = refs[i]; i += 1

        c = lax.axis_index("c")
        s = lax.axis_index("s")
        wid = s * NC + c

        pltpu.sync_copy(src_hbm.at[wid], src_v)
        pltpu.sync_copy(dst_hbm.at[wid], dst_v)

        z16 = jnp.zeros((16,), jnp.float32)

        if with_deg:
            o16 = jnp.ones((16,), jnp.float32)

            @pl.loop(0, K)
            def _fill(r):
                ones_v[r, pl.ds(0, 16)] = o16
                zb[r, pl.ds(0, 16)] = z16

            for t in range(ZROWS // K):
                pltpu.sync_copy(zb, accd.at[pl.ds(s * ZROWS + t * K, K)])

        for phase in range(nhalves):
            y_hbm = y_hbms[phase]
            out_hbm = out_hbms[phase]
            first = phase == 0

            @pl.loop(0, K)
            def _zero_rows(r):
                for q in range(d // 16):
                    rows[0][r, pl.ds(q * 16, 16)] = z16

            for t in range(ZROWS // K):
                pltpu.sync_copy(rows[0], acc.at[pl.ds(s * ZROWS + t * K, K)])
            plsc.subcore_barrier()

            for b in range(nbuf):
                pltpu.async_copy(y_hbm.at[src_v.at[b]], rows[b], gsem[b])
                if with_deg and first:
                    pltpu.async_copy(ones_v, accd.at[dst_v.at[b]], dsem[b],
                                     add=True)

            @pl.loop(0, CH // nbuf)
            def _rounds(t):
                jb = nbuf * t
                for b in range(nbuf):
                    pltpu.make_async_copy(
                        y_hbm.at[src_v.at[jb + b]], rows[b], gsem[b]).wait()
                    pltpu.async_copy(rows[b], acc.at[dst_v.at[jb + b]],
                                     ssem[b], add=True)
                for b in range(nbuf):
                    pltpu.make_async_copy(
                        rows[b], acc.at[dst_v.at[jb + b]], ssem[b]).wait()

                    @pl.when(t < CH // nbuf - 1)
                    def _prefetch(b=b):
                        pltpu.async_copy(
                            y_hbm.at[src_v.at[jb + nbuf + b]], rows[b],
                            gsem[b])

                if with_deg and first:
                    for b in range(nbuf):
                        pltpu.make_async_copy(
                            ones_v, accd.at[dst_v.at[jb + b]], dsem[b]).wait()

                        @pl.when(t < CH // nbuf - 1)
                        def _next_deg(b=b):
                            pltpu.async_copy(
                                ones_v, accd.at[dst_v.at[jb + nbuf + b]],
                                dsem[b], add=True)

            plsc.subcore_barrier()
            pltpu.sync_copy(acc.at[pl.ds(s * ZROWS, ZROWS)],
                            out_hbm.at[c, pl.ds(s * ZROWS, ZROWS)])
            if with_deg and first:
                pltpu.sync_copy(accd.at[pl.ds(s * ZROWS, ZROWS)],
                                outd_hbm.at[c, pl.ds(s * ZROWS, ZROWS)])
            if phase < nhalves - 1:
                plsc.subcore_barrier()

    return agg


_AGG1 = {32: _make_agg(32, 1, False, NBUF), 64: _make_agg(64, 1, False, NBUF)}
_AGG2_DEG = _make_agg(64, 2, True, 5)
_AGG2 = _make_agg(64, 2, False, NBUF)


def _nparts(do):
    return 2 if do >= 128 else 1


def _mm(h, Wn, Ws, b):
    di = h.shape[1]
    do = Wn.shape[1]
    nparts = _nparts(do)
    pw = do // nparts
    b2 = jnp.broadcast_to(b[None, :], (8, do))

    def body(h_ref, wn_ref, ws_ref, b_ref, *out_refs):
        hb = h_ref[...]
        wn = wn_ref[...]
        for i in range(nparts):
            out_refs[i][...] = jnp.dot(hb, wn[:, i * pw:(i + 1) * pw],
                                       preferred_element_type=jnp.float32)
        out_refs[nparts][...] = (
            jnp.dot(hb, ws_ref[...], preferred_element_type=jnp.float32)
            + b_ref[0:1, :])

    outs = pl.pallas_call(
        body,
        grid=(NPAD // MBLK,),
        in_specs=[
            pl.BlockSpec((MBLK, di), lambda i: (i, 0)),
            pl.BlockSpec((di, do), lambda i: (0, 0)),
            pl.BlockSpec((di, do), lambda i: (0, 0)),
            pl.BlockSpec((8, do), lambda i: (0, 0)),
        ],
        out_specs=[pl.BlockSpec((MBLK, pw), lambda i: (i, 0))
                   for _ in range(nparts)]
        + [pl.BlockSpec((MBLK, do), lambda i: (i, 0))],
        out_shape=[jax.ShapeDtypeStruct((NPAD, pw), jnp.float32)
                   for _ in range(nparts)]
        + [jax.ShapeDtypeStruct((NPAD, do), jnp.float32)],
    )(h, Wn, Ws, b2)
    return list(outs[:nparts]), outs[nparts]


def _fused(sv, ps, deg, Wn, Ws, b, extra=None, want_h=False):
    do = Wn.shape[1]
    nparts = _nparts(do)
    pw = do // nparts
    np_in = len(ps)
    has_extra = extra is not None
    hw = sv.shape[1] + (extra.shape[1] if has_extra else 0)
    b2 = jnp.broadcast_to(b[None, :], (8, do))

    def body(*refs):
        i = 0
        s_ref = refs[i]; i += 1
        p_refs = refs[i:i + np_in]; i += np_in
        dg_ref = refs[i]; i += 1
        e_ref = None
        if has_extra:
            e_ref = refs[i]; i += 1
        wn_ref = refs[i]; i += 1
        ws_ref = refs[i]; i += 1
        b_ref = refs[i]; i += 1
        out_refs = refs[i:]

        dg = dg_ref[0, :, 0:1] + dg_ref[1, :, 0:1]
        inv = 1.0 / jnp.maximum(dg, 1.0)
        psum = jnp.concatenate([pr[0] + pr[1] for pr in p_refs], axis=1)
        h = jnp.maximum(s_ref[...] + psum * inv, 0.0)
        if has_extra:
            h = jnp.concatenate([h, e_ref[...]], axis=1)
        wn = wn_ref[...]
        for j in range(nparts):
            out_refs[j][...] = jnp.dot(h, wn[:, j * pw:(j + 1) * pw],
                                       preferred_element_type=jnp.float32)
        out_refs[nparts][...] = (
            jnp.dot(h, ws_ref[...], preferred_element_type=jnp.float32)
            + b_ref[0:1, :])
        if want_h:
            out_refs[nparts + 1][...] = h

    in_specs = ([pl.BlockSpec((MBLK, sv.shape[1]), lambda i: (i, 0))]
                + [pl.BlockSpec((NC, MBLK, p.shape[2]), lambda i: (0, i, 0))
                   for p in ps]
                + [pl.BlockSpec((NC, MBLK, DEGW), lambda i: (0, i, 0))])
    args = [sv] + list(ps) + [deg]
    if has_extra:
        in_specs.append(pl.BlockSpec((MBLK, extra.shape[1]),
                                     lambda i: (i, 0)))
        args.append(extra)
    in_specs += [
        pl.BlockSpec((hw, do), lambda i: (0, 0)),
        pl.BlockSpec((hw, do), lambda i: (0, 0)),
        pl.BlockSpec((8, do), lambda i: (0, 0)),
    ]
    args += [Wn, Ws, b2]

    out_specs = ([pl.BlockSpec((MBLK, pw), lambda i: (i, 0))
                  for _ in range(nparts)]
                 + [pl.BlockSpec((MBLK, do), lambda i: (i, 0))])
    out_shape = ([jax.ShapeDtypeStruct((NPAD, pw), jnp.float32)
                  for _ in range(nparts)]
                 + [jax.ShapeDtypeStruct((NPAD, do), jnp.float32)])
    if want_h:
        out_specs.append(pl.BlockSpec((MBLK, hw), lambda i: (i, 0)))
        out_shape.append(jax.ShapeDtypeStruct((NPAD, hw), jnp.float32))

    outs = pl.pallas_call(
        body,
        grid=(NPAD // MBLK,),
        in_specs=in_specs,
        out_specs=out_specs,
        out_shape=out_shape,
    )(*args)
    yparts = list(outs[:nparts])
    if want_h:
        return yparts, outs[nparts], outs[nparts + 1]
    return yparts, outs[nparts]


def _combine(sv, ps, deg, relu):
    do = sv.shape[1]

    def body(s_ref, *refs):
        p_refs = refs[:len(ps)]
        dg_ref = refs[len(ps)]
        out_ref = refs[len(ps) + 1]
        dg = dg_ref[0, :, 0:1] + dg_ref[1, :, 0:1]
        inv = 1.0 / jnp.maximum(dg, 1.0)
        psum = jnp.concatenate([pr[0] + pr[1] for pr in p_refs], axis=1)
        r = s_ref[...] + psum * inv
        out_ref[...] = jnp.maximum(r, 0.0) if relu else r

    return pl.pallas_call(
        body,
        grid=(NPAD // MBLK,),
        in_specs=[pl.BlockSpec((MBLK, do), lambda i: (i, 0))]
        + [pl.BlockSpec((NC, MBLK, p.shape[2]), lambda i: (0, i, 0))
           for p in ps]
        + [pl.BlockSpec((NC, MBLK, DEGW), lambda i: (0, i, 0))],
        out_specs=pl.BlockSpec((MBLK, do), lambda i: (i, 0)),
        out_shape=jax.ShapeDtypeStruct((NPAD, do), jnp.float32),
    )(sv, *ps, deg)


def _agg_all(yparts, src3, dst3):
    if len(yparts) == 2:
        p0, p1 = _AGG2(yparts[0], yparts[1], src3, dst3)
        return [p0, p1]
    r = _AGG1[yparts[0].shape[1]](yparts[0], src3, dst3)
    return [r[0]] if isinstance(r, (list, tuple)) else [r]


def kernel(inputs, edge_index, W1s, W1n, b1, W2s, W2n, b2, W3s, W3n, b3,
           W4s, W4n, b4, W5s, W5n, b5, W6s, W6n, b6):
    x = inputs[0]
    x = jnp.pad(x, ((0, NPAD - N), (0, 0)))

    src = edge_index[0]
    dst = edge_index[1]
    npad_e = EPAD - E
    pad_src = (jnp.arange(npad_e, dtype=jnp.int32) % N)
    pad_dst = N + (jnp.arange(npad_e, dtype=jnp.int32) % (NPAD - N))
    src3 = jnp.concatenate([src, pad_src]).reshape(NW, CH, K)
    dst3 = jnp.concatenate([dst, pad_dst]).reshape(NW, CH, K)

    y1, s1 = _mm(x, W1n, W1s, b1)
    p1_lo, p1_hi, deg = _AGG2_DEG(y1[0], y1[1], src3, dst3)
    p1 = [p1_lo, p1_hi]
    y2, s2 = _fused(s1, p1, deg, W2n, W2s, b2)
    p2 = _agg_all(y2, src3, dst3)
    y3, s3, h2 = _fused(s2, p2, deg, W3n, W3s, b3, want_h=True)
    p3 = _agg_all(y3, src3, dst3)
    y4, s4, h3 = _fused(s3, p3, deg, W4n, W4s, b4, want_h=True)
    p4 = _agg_all(y4, src3, dst3)
    y5, s5 = _fused(s4, p4, deg, W4n, W4s, b4)
    p5 = _agg_all(y5, src3, dst3)
    y6, s6 = _fused(s5, p5, deg, W5n, W5s, b5, extra=h3)
    p6 = _agg_all(y6, src3, dst3)
    y7, s7 = _fused(s6, p6, deg, W6n, W6s, b6, extra=h2)
    p7 = _agg_all(y7, src3, dst3)
    out = _combine(s7, p7, deg, False)

    return out[:N][None, :, :]


# speedup vs baseline: 1.1006x; 1.1006x over previous
import functools

import jax
import jax.numpy as jnp
from jax import lax
from jax.experimental import pallas as pl
from jax.experimental.pallas import tpu as pltpu
from jax.experimental.pallas import tpu_sc as plsc

N = 10000
E = 320000
NPAD = 10240
NC = 2
NS = 16
NW = NC * NS
K = 128
CH = 80
EPAD = NW * CH * K
ZROWS = NPAD // NS
DEGW = 16
NBUF = 8

MBLK = 2000


def _make_agg(d, nhalves, with_deg, nbuf):
    mesh = plsc.VectorSubcoreMesh(core_axis_name="c", subcore_axis_name="s")

    out_type = [jax.ShapeDtypeStruct((NC, NPAD, d), jnp.float32)
                for _ in range(nhalves)]
    scratch = [
        pltpu.VMEM((CH, K), jnp.int32),
        pltpu.VMEM((CH, K), jnp.int32),
        [pltpu.VMEM((K, d), jnp.float32) for _ in range(nbuf)],
        [pltpu.SemaphoreType.DMA for _ in range(nbuf)],
        [pltpu.SemaphoreType.DMA for _ in range(nbuf)],
        pltpu.VMEM_SHARED((NPAD, d), jnp.float32),
    ]
    if with_deg:
        out_type.append(jax.ShapeDtypeStruct((NC, NPAD, DEGW), jnp.float32))
        scratch += [
            pltpu.VMEM((K, DEGW), jnp.float32),
            pltpu.VMEM((K, DEGW), jnp.float32),
            [pltpu.SemaphoreType.DMA for _ in range(nbuf)],
            pltpu.VMEM_SHARED((NPAD, DEGW), jnp.float32),
        ]

    @functools.partial(
        pl.kernel,
        out_type=out_type,
        mesh=mesh,
        compiler_params=pltpu.CompilerParams(use_tc_tiling_on_sc=False),
        scratch_types=scratch,
    )
    def agg(*refs):
        i = 0
        y_hbms = refs[i:i + nhalves]; i += nhalves
        src_hbm = refs[i]; i += 1
        dst_hbm = refs[i]; i += 1
        out_hbms = refs[i:i + nhalves]; i += nhalves
        if with_deg:
            outd_hbm = refs[i]; i += 1
        src_v = refs[i]; i += 1
        dst_v = refs[i]; i += 1
        rows = refs[i]; i += 1
        gsem = refs[i]; i += 1
        ssem = refs[i]; i += 1
        acc = refs[i]; i += 1
        if with_deg:
            ones_v = refs[i]; i += 1
            zb = refs[i]; i += 1
            dsem = refs[i]; i += 1
            accd = refs[i]; i += 1

        c = lax.axis_index("c")
        s = lax.axis_index("s")
        wid = s * NC + c

        pltpu.sync_copy(src_hbm.at[wid], src_v)
        pltpu.sync_copy(dst_hbm.at[wid], dst_v)

        z16 = jnp.zeros((16,), jnp.float32)

        if with_deg:
            o16 = jnp.ones((16,), jnp.float32)

            @pl.loop(0, K)
            def _fill(r):
                ones_v[r, pl.ds(0, 16)] = o16
                zb[r, pl.ds(0, 16)] = z16

            for t in range(ZROWS // K):
                pltpu.sync_copy(zb, accd.at[pl.ds(s * ZROWS + t * K, K)])

        for phase in range(nhalves):
            y_hbm = y_hbms[phase]
            out_hbm = out_hbms[phase]
            first = phase == 0

            @pl.loop(0, K)
            def _zero_rows(r):
                for q in range(d // 16):
                    rows[0][r, pl.ds(q * 16, 16)] = z16

            for t in range(ZROWS // K):
                pltpu.sync_copy(rows[0], acc.at[pl.ds(s * ZROWS + t * K, K)])
            plsc.subcore_barrier()

            for b in range(nbuf):
                pltpu.async_copy(y_hbm.at[src_v.at[b]], rows[b], gsem[b])
                if with_deg and first:
                    pltpu.async_copy(ones_v, accd.at[dst_v.at[b]], dsem[b],
                                     add=True)

            @pl.loop(0, CH // nbuf)
            def _rounds(t):
                jb = nbuf * t
                for b in range(nbuf):
                    pltpu.make_async_copy(
                        y_hbm.at[src_v.at[jb + b]], rows[b], gsem[b]).wait()
                    pltpu.async_copy(rows[b], acc.at[dst_v.at[jb + b]],
                                     ssem[b], add=True)
                for b in range(nbuf):
                    pltpu.make_async_copy(
                        rows[b], acc.at[dst_v.at[jb + b]], ssem[b]).wait()

                    @pl.when(t < CH // nbuf - 1)
                    def _prefetch(b=b):
                        pltpu.async_copy(
                            y_hbm.at[src_v.at[jb + nbuf + b]], rows[b],
                            gsem[b])

                if with_deg and first:
                    for b in range(nbuf):
                        pltpu.make_async_copy(
                            ones_v, accd.at[dst_v.at[jb + b]], dsem[b]).wait()

                        @pl.when(t < CH // nbuf - 1)
                        def _next_deg(b=b):
                            pltpu.async_copy(
                                ones_v, accd.at[dst_v.at[jb + nbuf + b]],
                                dsem[b], add=True)

            plsc.subcore_barrier()
            pltpu.sync_copy(acc.at[pl.ds(s * ZROWS, ZROWS)],
                            out_hbm.at[c, pl.ds(s * ZROWS, ZROWS)])
            if with_deg and first:
                pltpu.sync_copy(accd.at[pl.ds(s * ZROWS, ZROWS)],
                                outd_hbm.at[c, pl.ds(s * ZROWS, ZROWS)])
            if phase < nhalves - 1:
                plsc.subcore_barrier()

    return agg


_AGG1 = {32: _make_agg(32, 1, False, NBUF), 64: _make_agg(64, 1, False, NBUF)}
_AGG2_DEG = _make_agg(64, 2, True, 5)
_AGG2 = _make_agg(64, 2, False, NBUF)


def _nparts(do):
    return 2 if do >= 128 else 1


def _mm(h, Wn, Ws, b):
    di = h.shape[1]
    do = Wn.shape[1]
    nparts = _nparts(do)
    pw = do // nparts
    b2 = jnp.broadcast_to(b[None, :], (8, do))

    def body(h_ref, wn_ref, ws_ref, b_ref, *out_refs):
        hb = h_ref[...]
        wn = wn_ref[...]
        for i in range(nparts):
            out_refs[i][...] = jnp.dot(hb, wn[:, i * pw:(i + 1) * pw],
                                       preferred_element_type=jnp.float32)
        out_refs[nparts][...] = (
            jnp.dot(hb, ws_ref[...], preferred_element_type=jnp.float32)
            + b_ref[0:1, :])

    outs = pl.pallas_call(
        body,
        grid=(N // MBLK,),
        in_specs=[
            pl.BlockSpec((MBLK, di), lambda i: (i, 0)),
            pl.BlockSpec((di, do), lambda i: (0, 0)),
            pl.BlockSpec((di, do), lambda i: (0, 0)),
            pl.BlockSpec((8, do), lambda i: (0, 0)),
        ],
        out_specs=[pl.BlockSpec((MBLK, pw), lambda i: (i, 0))
                   for _ in range(nparts)]
        + [pl.BlockSpec((MBLK, do), lambda i: (i, 0))],
        out_shape=[jax.ShapeDtypeStruct((N, pw), jnp.float32)
                   for _ in range(nparts)]
        + [jax.ShapeDtypeStruct((N, do), jnp.float32)],
    )(h, Wn, Ws, b2)
    return list(outs[:nparts]), outs[nparts]


def _fused(sv, ps, deg, Wn, Ws, b, extra=None, want_h=False):
    do = Wn.shape[1]
    nparts = _nparts(do)
    pw = do // nparts
    np_in = len(ps)
    has_extra = extra is not None
    hw = sv.shape[1] + (extra.shape[1] if has_extra else 0)
    b2 = jnp.broadcast_to(b[None, :], (8, do))

    def body(*refs):
        i = 0
        s_ref = refs[i]; i += 1
        p_refs = refs[i:i + np_in]; i += np_in
        dg_ref = refs[i]; i += 1
        e_ref = None
        if has_extra:
            e_ref = refs[i]; i += 1
        wn_ref = refs[i]; i += 1
        ws_ref = refs[i]; i += 1
        b_ref = refs[i]; i += 1
        out_refs = refs[i:]

        dg = dg_ref[0, :, 0:1] + dg_ref[1, :, 0:1]
        inv = 1.0 / jnp.maximum(dg, 1.0)
        psum = jnp.concatenate([pr[0] + pr[1] for pr in p_refs], axis=1)
        h = jnp.maximum(s_ref[...] + psum * inv, 0.0)
        if has_extra:
            h = jnp.concatenate([h, e_ref[...]], axis=1)
        wn = wn_ref[...]
        for j in range(nparts):
            out_refs[j][...] = jnp.dot(h, wn[:, j * pw:(j + 1) * pw],
                                       preferred_element_type=jnp.float32)
        out_refs[nparts][...] = (
            jnp.dot(h, ws_ref[...], preferred_element_type=jnp.float32)
            + b_ref[0:1, :])
        if want_h:
            out_refs[nparts + 1][...] = h

    in_specs = ([pl.BlockSpec((MBLK, sv.shape[1]), lambda i: (i, 0))]
                + [pl.BlockSpec((NC, MBLK, p.shape[2]), lambda i: (0, i, 0))
                   for p in ps]
                + [pl.BlockSpec((NC, MBLK, DEGW), lambda i: (0, i, 0))])
    args = [sv] + list(ps) + [deg]
    if has_extra:
        in_specs.append(pl.BlockSpec((MBLK, extra.shape[1]),
                                     lambda i: (i, 0)))
        args.append(extra)
    in_specs += [
        pl.BlockSpec((hw, do), lambda i: (0, 0)),
        pl.BlockSpec((hw, do), lambda i: (0, 0)),
        pl.BlockSpec((8, do), lambda i: (0, 0)),
    ]
    args += [Wn, Ws, b2]

    out_specs = ([pl.BlockSpec((MBLK, pw), lambda i: (i, 0))
                  for _ in range(nparts)]
                 + [pl.BlockSpec((MBLK, do), lambda i: (i, 0))])
    out_shape = ([jax.ShapeDtypeStruct((N, pw), jnp.float32)
                  for _ in range(nparts)]
                 + [jax.ShapeDtypeStruct((N, do), jnp.float32)])
    if want_h:
        out_specs.append(pl.BlockSpec((MBLK, hw), lambda i: (i, 0)))
        out_shape.append(jax.ShapeDtypeStruct((N, hw), jnp.float32))

    outs = pl.pallas_call(
        body,
        grid=(N // MBLK,),
        in_specs=in_specs,
        out_specs=out_specs,
        out_shape=out_shape,
    )(*args)
    yparts = list(outs[:nparts])
    if want_h:
        return yparts, outs[nparts], outs[nparts + 1]
    return yparts, outs[nparts]


def _combine(sv, ps, deg, relu):
    do = sv.shape[1]

    def body(s_ref, *refs):
        p_refs = refs[:len(ps)]
        dg_ref = refs[len(ps)]
        out_ref = refs[len(ps) + 1]
        dg = dg_ref[0, :, 0:1] + dg_ref[1, :, 0:1]
        inv = 1.0 / jnp.maximum(dg, 1.0)
        psum = jnp.concatenate([pr[0] + pr[1] for pr in p_refs], axis=1)
        r = s_ref[...] + psum * inv
        out_ref[...] = jnp.maximum(r, 0.0) if relu else r

    return pl.pallas_call(
        body,
        grid=(N // MBLK,),
        in_specs=[pl.BlockSpec((MBLK, do), lambda i: (i, 0))]
        + [pl.BlockSpec((NC, MBLK, p.shape[2]), lambda i: (0, i, 0))
           for p in ps]
        + [pl.BlockSpec((NC, MBLK, DEGW), lambda i: (0, i, 0))],
        out_specs=pl.BlockSpec((MBLK, do), lambda i: (i, 0)),
        out_shape=jax.ShapeDtypeStruct((N, do), jnp.float32),
    )(sv, *ps, deg)


def _agg_all(yparts, src3, dst3):
    if len(yparts) == 2:
        p0, p1 = _AGG2(yparts[0], yparts[1], src3, dst3)
        return [p0, p1]
    r = _AGG1[yparts[0].shape[1]](yparts[0], src3, dst3)
    return [r[0]] if isinstance(r, (list, tuple)) else [r]


def kernel(inputs, edge_index, W1s, W1n, b1, W2s, W2n, b2, W3s, W3n, b3,
           W4s, W4n, b4, W5s, W5n, b5, W6s, W6n, b6):
    x = inputs[0]

    src = edge_index[0]
    dst = edge_index[1]
    npad_e = EPAD - E
    pad_src = (jnp.arange(npad_e, dtype=jnp.int32) % N)
    pad_dst = N + (jnp.arange(npad_e, dtype=jnp.int32) % (NPAD - N))
    src3 = jnp.concatenate([src, pad_src]).reshape(NW, CH, K)
    dst3 = jnp.concatenate([dst, pad_dst]).reshape(NW, CH, K)

    y1, s1 = _mm(x, W1n, W1s, b1)
    p1_lo, p1_hi, deg = _AGG2_DEG(y1[0], y1[1], src3, dst3)
    p1 = [p1_lo, p1_hi]
    y2, s2 = _fused(s1, p1, deg, W2n, W2s, b2)
    p2 = _agg_all(y2, src3, dst3)
    y3, s3, h2 = _fused(s2, p2, deg, W3n, W3s, b3, want_h=True)
    p3 = _agg_all(y3, src3, dst3)
    y4, s4, h3 = _fused(s3, p3, deg, W4n, W4s, b4, want_h=True)
    p4 = _agg_all(y4, src3, dst3)
    y5, s5 = _fused(s4, p4, deg, W4n, W4s, b4)
    p5 = _agg_all(y5, src3, dst3)
    y6, s6 = _fused(s5, p5, deg, W5n, W5s, b5, extra=h3)
    p6 = _agg_all(y6, src3, dst3)
    y7, s7 = _fused(s6, p6, deg, W6n, W6s, b6, extra=h2)
    p7 = _agg_all(y7, src3, dst3)
    out = _combine(s7, p7, deg, False)

    return out[None, :, :]
